# baseline (speedup 1.0000x reference)
import functools

import jax
import jax.numpy as jnp
from jax import lax
from jax.experimental import pallas as pl
from jax.experimental.pallas import tpu as pltpu
from jax.experimental.pallas import tpu_sc as plsc

N_TOKENS = 8192
D_MODEL = 2048
NUM_EXPERTS = 16
TOP_K = 2

N_STREAMS = 4
BLOCK_T = 2048
HALF = BLOCK_T // N_STREAMS

NC, NS, LANES = 2, 16, 16
NW = NC * NS
T_PER_W = N_TOKENS // NW
GROUPS = T_PER_W // LANES


def _noisy_half(xb, w, b, epsb):
    both = jnp.dot(xb, w, preferred_element_type=jnp.float32) + b
    logits = both[:, :NUM_EXPERTS]
    nlogits = both[:, NUM_EXPERTS:]
    return logits + epsb * jax.nn.softplus(nlogits)


def _noisy_body(*refs):
    x_refs = refs[:N_STREAMS]
    w_ref, b_ref, eps_ref, noisy_ref = refs[N_STREAMS:]
    w = w_ref[...]
    b = b_ref[...]
    for s in range(N_STREAMS):
        lo = s * HALF
        noisy_ref[pl.ds(lo, HALF), :] = _noisy_half(
            x_refs[s][...], w, b, eps_ref[pl.ds(lo, HALF), :])


def _x_spec(s):
    return pl.BlockSpec((HALF, D_MODEL), lambda i: (N_STREAMS * i + s, 0))


def _noisy_logits(x, w_cat, b_cat, eps):
    n_blocks = N_TOKENS // BLOCK_T
    return pl.pallas_call(
        _noisy_body,
        grid=(n_blocks,),
        in_specs=[_x_spec(s) for s in range(N_STREAMS)] + [
            pl.BlockSpec((D_MODEL, 2 * NUM_EXPERTS), lambda i: (0, 0)),
            pl.BlockSpec((1, 2 * NUM_EXPERTS), lambda i: (0, 0)),
            pl.BlockSpec((BLOCK_T, NUM_EXPERTS), lambda i: (i, 0)),
        ],
        out_specs=pl.BlockSpec((BLOCK_T, NUM_EXPERTS), lambda i: (i, 0)),
        out_shape=jax.ShapeDtypeStruct((N_TOKENS, NUM_EXPERTS),
                                       jnp.float32),
    )(*([x] * N_STREAMS), w_cat, b_cat, eps)


@functools.cache
def _make_sc_topk_softmax():
    return pl.kernel(
        _sc_topk_softmax_body,
        mesh=plsc.VectorSubcoreMesh(core_axis_name="c",
                                    subcore_axis_name="s"),
        compiler_params=pltpu.CompilerParams(needs_layout_passes=False),
        out_type=(
            jax.ShapeDtypeStruct((N_TOKENS * NUM_EXPERTS,), jnp.float32),
            jax.ShapeDtypeStruct((N_TOKENS * TOP_K,), jnp.int32),
        ),
        scratch_types=[
            pltpu.VMEM((T_PER_W * NUM_EXPERTS,), jnp.float32),
            pltpu.VMEM((T_PER_W * NUM_EXPERTS,), jnp.float32),
            pltpu.VMEM((T_PER_W * TOP_K,), jnp.int32),
        ],
    )


def _sc_topk_softmax_body(noisy_hbm, out_hbm, idx_hbm, rows_v, out_v, idx_v):
    wid = lax.axis_index("s") * NC + lax.axis_index("c")
    pltpu.sync_copy(
        noisy_hbm.at[pl.ds(wid * T_PER_W * NUM_EXPERTS,
                           T_PER_W * NUM_EXPERTS)], rows_v)

    lanes = lax.iota(jnp.int32, LANES)
    neg_inf = jnp.full((LANES,), -jnp.inf, jnp.float32)
    zero = jnp.zeros((LANES,), jnp.float32)

    def group(g, carry):
        tok = lanes + g * LANES
        tok_e = tok * NUM_EXPERTS
        tok_k = tok * TOP_K
        cols = [
            plsc.load_gather(rows_v, [tok_e + e])
            for e in range(NUM_EXPERTS)
        ]
        m1 = cols[0]
        i1 = jnp.zeros((LANES,), jnp.int32)
        for e in range(1, NUM_EXPERTS):
            upd = cols[e] > m1
            m1 = jnp.where(upd, cols[e], m1)
            i1 = jnp.where(upd, jnp.full((LANES,), e, jnp.int32), i1)
        m2 = neg_inf
        i2 = jnp.full((LANES,), NUM_EXPERTS, jnp.int32)
        for e in range(NUM_EXPERTS):
            evec = jnp.full((LANES,), e, jnp.int32)
            cand = jnp.where(i1 == evec, neg_inf, cols[e])
            upd = cand > m2
            m2 = jnp.where(upd, cand, m2)
            i2 = jnp.where(upd, evec, i2)
        zs = []
        for e in range(NUM_EXPERTS):
            evec = jnp.full((LANES,), e, jnp.int32)
            keep = (i1 == evec) | (i2 == evec)
            zs.append(jnp.where(keep, jnp.exp(cols[e] - m1), zero))
        total = zs[0]
        for e in range(1, NUM_EXPERTS):
            total = total + zs[e]
        for e in range(NUM_EXPERTS):
            plsc.store_scatter(out_v, [tok_e + e], zs[e] / total)
        plsc.store_scatter(idx_v, [tok_k], i1)
        plsc.store_scatter(idx_v, [tok_k + 1], i2)
        return carry

    lax.fori_loop(0, GROUPS, group, 0)
    pltpu.sync_copy(
        out_v, out_hbm.at[pl.ds(wid * T_PER_W * NUM_EXPERTS,
                                T_PER_W * NUM_EXPERTS)])
    pltpu.sync_copy(
        idx_v, idx_hbm.at[pl.ds(wid * T_PER_W * TOP_K,
                                T_PER_W * TOP_K)])


def kernel(x, W_route, b_route, W_noise, b_noise, eps):
    w_cat = jnp.concatenate([W_route, W_noise], axis=1)
    b_cat = jnp.concatenate([b_route, b_noise]).reshape(1, 2 * NUM_EXPERTS)
    noisy = _noisy_logits(x, w_cat, b_cat, eps)
    out_flat, idx_flat = _make_sc_topk_softmax()(
        noisy.reshape(N_TOKENS * NUM_EXPERTS))
    router_output = out_flat.reshape(N_TOKENS, NUM_EXPERTS)
    topk_indices = idx_flat.reshape(N_TOKENS, TOP_K)
    return (router_output, topk_indices)


# speedup vs baseline: 1.0165x; 1.0165x over previous
import functools

import jax
import jax.numpy as jnp
from jax import lax
from jax.experimental import pallas as pl
from jax.experimental.pallas import tpu as pltpu
from jax.experimental.pallas import tpu_sc as plsc

N_TOKENS = 8192
D_MODEL = 2048
NUM_EXPERTS = 16
TOP_K = 2

N_STREAMS = 4
BLOCK_T = 2048
HALF = BLOCK_T // N_STREAMS

SC_TOKENS = 2048
TC_TOKENS = N_TOKENS - SC_TOKENS

NC, NS, LANES = 2, 16, 16
NW = NC * NS
T_PER_W = SC_TOKENS // NW
GROUPS = T_PER_W // LANES


def _noisy_half(xb, w, b, epsb):
    both = jnp.dot(xb, w, preferred_element_type=jnp.float32) + b
    logits = both[:, :NUM_EXPERTS]
    nlogits = both[:, NUM_EXPERTS:]
    return logits + epsb * jax.nn.softplus(nlogits)


def _topk_softmax_tc(noisy):
    iota = lax.broadcasted_iota(jnp.int32, noisy.shape, 1)
    m1 = jnp.max(noisy, axis=1, keepdims=True)
    i1 = jnp.min(jnp.where(noisy == m1, iota, NUM_EXPERTS), axis=1,
                 keepdims=True)
    masked = jnp.where(iota == i1, -jnp.inf, noisy)
    m2 = jnp.max(masked, axis=1, keepdims=True)
    i2 = jnp.min(jnp.where(masked == m2, iota, NUM_EXPERTS), axis=1,
                 keepdims=True)
    keep = (iota == i1) | (iota == i2)
    z = jnp.where(keep, jnp.exp(noisy - m1), 0.0)
    out = z / jnp.sum(z, axis=1, keepdims=True)
    idx = jnp.concatenate([i1, i2], axis=1)
    return out, idx


def _noisy_body(*refs):
    x_refs = refs[:N_STREAMS]
    w_ref, b_ref, eps_ref, noisy_ref = refs[N_STREAMS:]
    w = w_ref[...]
    b = b_ref[...]
    for s in range(N_STREAMS):
        lo = s * HALF
        noisy_ref[pl.ds(lo, HALF), :] = _noisy_half(
            x_refs[s][...], w, b, eps_ref[pl.ds(lo, HALF), :])


def _fused_body(*refs):
    x_refs = refs[:N_STREAMS]
    w_ref, b_ref, eps_ref, out_ref, idx_ref = refs[N_STREAMS:]
    w = w_ref[...]
    b = b_ref[...]
    for s in range(N_STREAMS):
        lo = s * HALF
        noisy = _noisy_half(x_refs[s][...], w, b,
                            eps_ref[pl.ds(lo, HALF), :])
        out_s, idx_s = _topk_softmax_tc(noisy)
        out_ref[pl.ds(lo, HALF), :] = out_s
        idx_ref[pl.ds(lo, HALF), :] = idx_s


def _x_spec(s, blk_base):
    return pl.BlockSpec(
        (HALF, D_MODEL), lambda i: (blk_base + N_STREAMS * i + s, 0))


def _common_specs(t0):
    c0 = t0 // BLOCK_T
    return [
        pl.BlockSpec((D_MODEL, 2 * NUM_EXPERTS), lambda i: (0, 0)),
        pl.BlockSpec((1, 2 * NUM_EXPERTS), lambda i: (0, 0)),
        pl.BlockSpec((BLOCK_T, NUM_EXPERTS), lambda i: (c0 + i, 0)),
    ]


def _noisy_chunk(x, w_cat, b_cat, eps, t0, nt):
    blk_base = t0 // HALF
    return pl.pallas_call(
        _noisy_body,
        grid=(nt // BLOCK_T,),
        in_specs=[_x_spec(s, blk_base) for s in range(N_STREAMS)]
        + _common_specs(t0),
        out_specs=pl.BlockSpec((BLOCK_T, NUM_EXPERTS), lambda i: (i, 0)),
        out_shape=jax.ShapeDtypeStruct((nt, NUM_EXPERTS), jnp.float32),
    )(*([x] * N_STREAMS), w_cat, b_cat, eps)


def _fused_chunk(x, w_cat, b_cat, eps, t0, nt):
    blk_base = t0 // HALF
    return pl.pallas_call(
        _fused_body,
        grid=(nt // BLOCK_T,),
        in_specs=[_x_spec(s, blk_base) for s in range(N_STREAMS)]
        + _common_specs(t0),
        out_specs=(
            pl.BlockSpec((BLOCK_T, NUM_EXPERTS), lambda i: (i, 0)),
            pl.BlockSpec((BLOCK_T, TOP_K), lambda i: (i, 0)),
        ),
        out_shape=(
            jax.ShapeDtypeStruct((nt, NUM_EXPERTS), jnp.float32),
            jax.ShapeDtypeStruct((nt, TOP_K), jnp.int32),
        ),
    )(*([x] * N_STREAMS), w_cat, b_cat, eps)


@functools.cache
def _make_sc_topk_softmax():
    return pl.kernel(
        _sc_topk_softmax_body,
        mesh=plsc.VectorSubcoreMesh(core_axis_name="c",
                                    subcore_axis_name="s"),
        compiler_params=pltpu.CompilerParams(needs_layout_passes=False),
        out_type=(
            jax.ShapeDtypeStruct((SC_TOKENS * NUM_EXPERTS,), jnp.float32),
            jax.ShapeDtypeStruct((SC_TOKENS * TOP_K,), jnp.int32),
        ),
        scratch_types=[
            pltpu.VMEM((T_PER_W * NUM_EXPERTS,), jnp.float32),
            pltpu.VMEM((T_PER_W * NUM_EXPERTS,), jnp.float32),
            pltpu.VMEM((T_PER_W * TOP_K,), jnp.int32),
        ],
    )


def _sc_topk_softmax_body(noisy_hbm, out_hbm, idx_hbm, rows_v, out_v, idx_v):
    wid = lax.axis_index("s") * NC + lax.axis_index("c")
    pltpu.sync_copy(
        noisy_hbm.at[pl.ds(wid * T_PER_W * NUM_EXPERTS,
                           T_PER_W * NUM_EXPERTS)], rows_v)

    lanes = lax.iota(jnp.int32, LANES)
    neg_inf = jnp.full((LANES,), -jnp.inf, jnp.float32)
    zero = jnp.zeros((LANES,), jnp.float32)

    def group(g, carry):
        tok = lanes + g * LANES
        tok_e = tok * NUM_EXPERTS
        tok_k = tok * TOP_K
        cols = [
            plsc.load_gather(rows_v, [tok_e + e])
            for e in range(NUM_EXPERTS)
        ]
        m1 = cols[0]
        i1 = jnp.zeros((LANES,), jnp.int32)
        for e in range(1, NUM_EXPERTS):
            upd = cols[e] > m1
            m1 = jnp.where(upd, cols[e], m1)
            i1 = jnp.where(upd, jnp.full((LANES,), e, jnp.int32), i1)
        m2 = neg_inf
        i2 = jnp.full((LANES,), NUM_EXPERTS, jnp.int32)
        for e in range(NUM_EXPERTS):
            evec = jnp.full((LANES,), e, jnp.int32)
            cand = jnp.where(i1 == evec, neg_inf, cols[e])
            upd = cand > m2
            m2 = jnp.where(upd, cand, m2)
            i2 = jnp.where(upd, evec, i2)
        zs = []
        for e in range(NUM_EXPERTS):
            evec = jnp.full((LANES,), e, jnp.int32)
            keep = (i1 == evec) | (i2 == evec)
            zs.append(jnp.where(keep, jnp.exp(cols[e] - m1), zero))
        total = zs[0]
        for e in range(1, NUM_EXPERTS):
            total = total + zs[e]
        for e in range(NUM_EXPERTS):
            plsc.store_scatter(out_v, [tok_e + e], zs[e] / total)
        plsc.store_scatter(idx_v, [tok_k], i1)
        plsc.store_scatter(idx_v, [tok_k + 1], i2)
        return carry

    lax.fori_loop(0, GROUPS, group, 0)
    pltpu.sync_copy(
        out_v, out_hbm.at[pl.ds(wid * T_PER_W * NUM_EXPERTS,
                                T_PER_W * NUM_EXPERTS)])
    pltpu.sync_copy(
        idx_v, idx_hbm.at[pl.ds(wid * T_PER_W * TOP_K,
                                T_PER_W * TOP_K)])


def kernel(x, W_route, b_route, W_noise, b_noise, eps):
    w_cat = jnp.concatenate([W_route, W_noise], axis=1)
    b_cat = jnp.concatenate([b_route, b_noise]).reshape(1, 2 * NUM_EXPERTS)
    noisy0 = _noisy_chunk(x, w_cat, b_cat, eps, 0, SC_TOKENS)
    out0f, idx0f = _make_sc_topk_softmax()(
        noisy0.reshape(SC_TOKENS * NUM_EXPERTS))
    out1, idx1 = _fused_chunk(x, w_cat, b_cat, eps, SC_TOKENS, TC_TOKENS)
    router_output = jnp.concatenate(
        [out0f.reshape(SC_TOKENS, NUM_EXPERTS), out1], axis=0)
    topk_indices = jnp.concatenate(
        [idx0f.reshape(SC_TOKENS, TOP_K), idx1], axis=0)
    return (router_output, topk_indices)


# speedup vs baseline: 1.4974x; 1.4731x over previous
import jax
import jax.numpy as jnp
from jax import lax
from jax.experimental import pallas as pl

N_TOKENS = 8192
D_MODEL = 2048
NUM_EXPERTS = 16
TOP_K = 2

N_STREAMS = 4
BLOCK_T = 2048
HALF = BLOCK_T // N_STREAMS


def _route_half(xb, w, b, epsb):
    both = jnp.dot(xb, w, preferred_element_type=jnp.float32) + b
    logits = both[:, :NUM_EXPERTS]
    nlogits = both[:, NUM_EXPERTS:]
    noisy = logits + epsb * jax.nn.softplus(nlogits)

    iota = lax.broadcasted_iota(jnp.int32, noisy.shape, 1)
    m1 = jnp.max(noisy, axis=1, keepdims=True)
    i1 = jnp.min(jnp.where(noisy == m1, iota, NUM_EXPERTS), axis=1,
                 keepdims=True)
    masked = jnp.where(iota == i1, -jnp.inf, noisy)
    m2 = jnp.max(masked, axis=1, keepdims=True)
    i2 = jnp.min(jnp.where(masked == m2, iota, NUM_EXPERTS), axis=1,
                 keepdims=True)
    keep = (iota == i1) | (iota == i2)
    z = jnp.where(keep, jnp.exp(noisy - m1), 0.0)
    out = z / jnp.sum(z, axis=1, keepdims=True)
    idx = jnp.concatenate([i1, i2], axis=1)
    return out, idx


def _router_body(*refs):
    x_refs = refs[:N_STREAMS]
    wr_ref, wn_ref, br_ref, bn_ref, eps_ref, out_ref, idx_ref = \
        refs[N_STREAMS:]
    w = jnp.concatenate([wr_ref[...], wn_ref[...]], axis=1)
    b = jnp.concatenate([br_ref[...], bn_ref[...]], axis=1)
    for s in range(N_STREAMS):
        lo = s * HALF
        out_s, idx_s = _route_half(x_refs[s][...], w, b,
                                   eps_ref[pl.ds(lo, HALF), :])
        out_ref[pl.ds(lo, HALF), :] = out_s
        idx_ref[pl.ds(lo, HALF), :] = idx_s


def _x_spec(s):
    return pl.BlockSpec((HALF, D_MODEL), lambda i: (N_STREAMS * i + s, 0))


def kernel(x, W_route, b_route, W_noise, b_noise, eps):
    n_blocks = N_TOKENS // BLOCK_T
    br = b_route.reshape(1, NUM_EXPERTS)
    bn = b_noise.reshape(1, NUM_EXPERTS)
    out_shapes = (
        jax.ShapeDtypeStruct((N_TOKENS, NUM_EXPERTS), jnp.float32),
        jax.ShapeDtypeStruct((N_TOKENS, TOP_K), jnp.int32),
    )
    router_output, topk_indices = pl.pallas_call(
        _router_body,
        grid=(n_blocks,),
        in_specs=[_x_spec(s) for s in range(N_STREAMS)] + [
            pl.BlockSpec((D_MODEL, NUM_EXPERTS), lambda i: (0, 0)),
            pl.BlockSpec((D_MODEL, NUM_EXPERTS), lambda i: (0, 0)),
            pl.BlockSpec((1, NUM_EXPERTS), lambda i: (0, 0)),
            pl.BlockSpec((1, NUM_EXPERTS), lambda i: (0, 0)),
            pl.BlockSpec((BLOCK_T, NUM_EXPERTS), lambda i: (i, 0)),
        ],
        out_specs=(
            pl.BlockSpec((BLOCK_T, NUM_EXPERTS), lambda i: (i, 0)),
            pl.BlockSpec((BLOCK_T, TOP_K), lambda i: (i, 0)),
        ),
        out_shape=out_shapes,
    )(*([x] * N_STREAMS), W_route, W_noise, br, bn, eps)
    return (router_output, topk_indices)


# speedup vs baseline: 2.4402x; 1.6296x over previous
import jax
import jax.numpy as jnp
from jax import lax
from jax.experimental import pallas as pl

N_TOKENS = 8192
D_MODEL = 2048
NUM_EXPERTS = 16
TOP_K = 2

N_STREAMS = 4
BLOCK_T = 2048
HALF = BLOCK_T // N_STREAMS

_NT = (((1,), (1,)), ((), ()))


def _route_stream(xb, w_t, b_col, eps_blk):
    both_t = lax.dot_general(w_t, xb, _NT,
                             preferred_element_type=jnp.float32)
    both_t = both_t + b_col
    logits_t = both_t[:NUM_EXPERTS, :]
    nlogits_t = both_t[NUM_EXPERTS:, :]
    noisy = logits_t + eps_blk * jax.nn.softplus(nlogits_t)

    iota = lax.broadcasted_iota(jnp.int32, noisy.shape, 0)
    m1 = jnp.max(noisy, axis=0, keepdims=True)
    i1 = jnp.min(jnp.where(noisy == m1, iota, NUM_EXPERTS), axis=0,
                 keepdims=True)
    masked = jnp.where(iota == i1, -jnp.inf, noisy)
    m2 = jnp.max(masked, axis=0, keepdims=True)
    i2 = jnp.min(jnp.where(masked == m2, iota, NUM_EXPERTS), axis=0,
                 keepdims=True)
    keep = (iota == i1) | (iota == i2)
    z = jnp.where(keep, jnp.exp(noisy - m1), 0.0)
    out = z / jnp.sum(z, axis=0, keepdims=True)
    idx = jnp.concatenate([i1, i2], axis=0)
    return out, idx


def _router_body(*refs):
    x_refs = refs[:N_STREAMS]
    wr_ref, wn_ref, br_ref, bn_ref, eps_ref, out_ref, idx_ref = \
        refs[N_STREAMS:]
    w_t = jnp.concatenate([wr_ref[...], wn_ref[...]], axis=0)
    b_col = jnp.concatenate([br_ref[...], bn_ref[...]], axis=0)
    for s in range(N_STREAMS):
        lo = s * HALF
        out_s, idx_s = _route_stream(x_refs[s][...], w_t, b_col,
                                     eps_ref[:, pl.ds(lo, HALF)])
        out_ref[:, pl.ds(lo, HALF)] = out_s
        idx_ref[:, pl.ds(lo, HALF)] = idx_s


def _x_spec(s):
    return pl.BlockSpec((HALF, D_MODEL), lambda i: (N_STREAMS * i + s, 0))


def kernel(x, W_route, b_route, W_noise, b_noise, eps):
    n_blocks = N_TOKENS // BLOCK_T
    wr_t = W_route.T
    wn_t = W_noise.T
    br = b_route.reshape(NUM_EXPERTS, 1)
    bn = b_noise.reshape(NUM_EXPERTS, 1)
    eps_t = eps.T
    out_shapes = (
        jax.ShapeDtypeStruct((NUM_EXPERTS, N_TOKENS), jnp.float32),
        jax.ShapeDtypeStruct((TOP_K, N_TOKENS), jnp.int32),
    )
    out_t, idx_t = pl.pallas_call(
        _router_body,
        grid=(n_blocks,),
        in_specs=[_x_spec(s) for s in range(N_STREAMS)] + [
            pl.BlockSpec((NUM_EXPERTS, D_MODEL), lambda i: (0, 0)),
            pl.BlockSpec((NUM_EXPERTS, D_MODEL), lambda i: (0, 0)),
            pl.BlockSpec((NUM_EXPERTS, 1), lambda i: (0, 0)),
            pl.BlockSpec((NUM_EXPERTS, 1), lambda i: (0, 0)),
            pl.BlockSpec((NUM_EXPERTS, BLOCK_T), lambda i: (0, i)),
        ],
        out_specs=(
            pl.BlockSpec((NUM_EXPERTS, BLOCK_T), lambda i: (0, i)),
            pl.BlockSpec((TOP_K, BLOCK_T), lambda i: (0, i)),
        ),
        out_shape=out_shapes,
    )(*([x] * N_STREAMS), wr_t, wn_t, br, bn, eps_t)
    return (out_t.T, idx_t.T)


# speedup vs baseline: 2.7052x; 1.1086x over previous
import jax
import jax.numpy as jnp
from jax import lax
from jax.experimental import pallas as pl

N_TOKENS = 8192
D_MODEL = 2048
NUM_EXPERTS = 16
TOP_K = 2

N_STREAMS = 4
BLOCK_T = 2048
HALF = BLOCK_T // N_STREAMS

_NT = (((1,), (1,)), ((), ()))


def _route_stream(xb, w_t, b_col, eps_blk):
    both_t = lax.dot_general(w_t, xb, _NT,
                             preferred_element_type=jnp.float32)
    both_t = both_t + b_col
    logits_t = both_t[:NUM_EXPERTS, :]
    nlogits_t = both_t[NUM_EXPERTS:, :]
    noisy = logits_t + eps_blk * jax.nn.softplus(nlogits_t)

    iota = lax.broadcasted_iota(jnp.int32, noisy.shape, 0)
    m1 = jnp.max(noisy, axis=0, keepdims=True)
    i1 = jnp.min(jnp.where(noisy == m1, iota, NUM_EXPERTS), axis=0,
                 keepdims=True)
    masked = jnp.where(iota == i1, -jnp.inf, noisy)
    m2 = jnp.max(masked, axis=0, keepdims=True)
    i2 = jnp.min(jnp.where(masked == m2, iota, NUM_EXPERTS), axis=0,
                 keepdims=True)
    keep = (iota == i1) | (iota == i2)
    z = jnp.where(keep, jnp.exp(noisy - m1), 0.0)
    out = z / jnp.sum(z, axis=0, keepdims=True)
    idx = jnp.concatenate([i1, i2], axis=0)
    return out, idx


def _router_body(*refs):
    x_refs = refs[:N_STREAMS]
    wr_ref, wn_ref, br_ref, bn_ref, eps_ref, out_ref, idx_ref = \
        refs[N_STREAMS:]
    w_t = jnp.concatenate([wr_ref[...], wn_ref[...]], axis=0)
    b_col = jnp.concatenate([br_ref[...], bn_ref[...]], axis=1).T
    for s in range(N_STREAMS):
        lo = s * HALF
        out_s, idx_s = _route_stream(x_refs[s][...], w_t, b_col,
                                     eps_ref[:, pl.ds(lo, HALF)])
        out_ref[:, pl.ds(lo, HALF)] = out_s
        idx_ref[:, pl.ds(lo, HALF)] = idx_s


def _x_spec(s):
    return pl.BlockSpec((HALF, D_MODEL), lambda i: (N_STREAMS * i + s, 0))


def kernel(x, W_route, b_route, W_noise, b_noise, eps):
    n_blocks = N_TOKENS // BLOCK_T
    wr_t = W_route.T
    wn_t = W_noise.T
    br = b_route.reshape(1, NUM_EXPERTS)
    bn = b_noise.reshape(1, NUM_EXPERTS)
    eps_t = eps.T
    out_shapes = (
        jax.ShapeDtypeStruct((NUM_EXPERTS, N_TOKENS), jnp.float32),
        jax.ShapeDtypeStruct((TOP_K, N_TOKENS), jnp.int32),
    )
    out_t, idx_t = pl.pallas_call(
        _router_body,
        grid=(n_blocks,),
        in_specs=[_x_spec(s) for s in range(N_STREAMS)] + [
            pl.BlockSpec((NUM_EXPERTS, D_MODEL), lambda i: (0, 0)),
            pl.BlockSpec((NUM_EXPERTS, D_MODEL), lambda i: (0, 0)),
            pl.BlockSpec((1, NUM_EXPERTS), lambda i: (0, 0)),
            pl.BlockSpec((1, NUM_EXPERTS), lambda i: (0, 0)),
            pl.BlockSpec((NUM_EXPERTS, BLOCK_T), lambda i: (0, i)),
        ],
        out_specs=(
            pl.BlockSpec((NUM_EXPERTS, BLOCK_T), lambda i: (0, i)),
            pl.BlockSpec((TOP_K, BLOCK_T), lambda i: (0, i)),
        ),
        out_shape=out_shapes,
    )(*([x] * N_STREAMS), wr_t, wn_t, br, bn, eps_t)
    return (out_t.T, idx_t.T)


# speedup vs baseline: 2.7137x; 1.0031x over previous
import jax
import jax.numpy as jnp
from jax import lax
from jax.experimental import pallas as pl

N_TOKENS = 8192
D_MODEL = 2048
NUM_EXPERTS = 16
TOP_K = 2

N_STREAMS = 8
BLOCK_T = 2048
HALF = BLOCK_T // N_STREAMS

_NT = (((1,), (1,)), ((), ()))


def _route_stream(xb, w_t, b_col, eps_blk):
    both_t = lax.dot_general(w_t, xb, _NT,
                             preferred_element_type=jnp.float32)
    both_t = both_t + b_col
    logits_t = both_t[:NUM_EXPERTS, :]
    nlogits_t = both_t[NUM_EXPERTS:, :]
    noisy = logits_t + eps_blk * jax.nn.softplus(nlogits_t)

    iota = lax.broadcasted_iota(jnp.int32, noisy.shape, 0)
    m1 = jnp.max(noisy, axis=0, keepdims=True)
    i1 = jnp.min(jnp.where(noisy == m1, iota, NUM_EXPERTS), axis=0,
                 keepdims=True)
    masked = jnp.where(iota == i1, -jnp.inf, noisy)
    m2 = jnp.max(masked, axis=0, keepdims=True)
    i2 = jnp.min(jnp.where(masked == m2, iota, NUM_EXPERTS), axis=0,
                 keepdims=True)
    keep = (iota == i1) | (iota == i2)
    z = jnp.where(keep, jnp.exp(noisy - m1), 0.0)
    out = z / jnp.sum(z, axis=0, keepdims=True)
    idx = jnp.concatenate([i1, i2], axis=0)
    return out, idx


def _router_body(*refs):
    x_refs = refs[:N_STREAMS]
    wr_ref, wn_ref, br_ref, bn_ref, eps_ref, out_ref, idx_ref = \
        refs[N_STREAMS:]
    w_t = jnp.concatenate([wr_ref[...], wn_ref[...]], axis=0)
    b_col = jnp.concatenate([br_ref[...], bn_ref[...]], axis=1).T
    for s in range(N_STREAMS):
        lo = s * HALF
        out_s, idx_s = _route_stream(x_refs[s][...], w_t, b_col,
                                     eps_ref[:, pl.ds(lo, HALF)])
        out_ref[:, pl.ds(lo, HALF)] = out_s
        idx_ref[:, pl.ds(lo, HALF)] = idx_s


def _x_spec(s):
    return pl.BlockSpec((HALF, D_MODEL), lambda i: (N_STREAMS * i + s, 0))


def kernel(x, W_route, b_route, W_noise, b_noise, eps):
    n_blocks = N_TOKENS // BLOCK_T
    wr_t = W_route.T
    wn_t = W_noise.T
    br = b_route.reshape(1, NUM_EXPERTS)
    bn = b_noise.reshape(1, NUM_EXPERTS)
    eps_t = eps.T
    out_shapes = (
        jax.ShapeDtypeStruct((NUM_EXPERTS, N_TOKENS), jnp.float32),
        jax.ShapeDtypeStruct((TOP_K, N_TOKENS), jnp.int32),
    )
    out_t, idx_t = pl.pallas_call(
        _router_body,
        grid=(n_blocks,),
        in_specs=[_x_spec(s) for s in range(N_STREAMS)] + [
            pl.BlockSpec((NUM_EXPERTS, D_MODEL), lambda i: (0, 0)),
            pl.BlockSpec((NUM_EXPERTS, D_MODEL), lambda i: (0, 0)),
            pl.BlockSpec((1, NUM_EXPERTS), lambda i: (0, 0)),
            pl.BlockSpec((1, NUM_EXPERTS), lambda i: (0, 0)),
            pl.BlockSpec((NUM_EXPERTS, BLOCK_T), lambda i: (0, i)),
        ],
        out_specs=(
            pl.BlockSpec((NUM_EXPERTS, BLOCK_T), lambda i: (0, i)),
            pl.BlockSpec((TOP_K, BLOCK_T), lambda i: (0, i)),
        ),
        out_shape=out_shapes,
    )(*([x] * N_STREAMS), wr_t, wn_t, br, bn, eps_t)
    return (out_t.T, idx_t.T)


# speedup vs baseline: 2.8145x; 1.0371x over previous
import jax
import jax.numpy as jnp
from jax import lax
from jax.experimental import pallas as pl

N_TOKENS = 8192
D_MODEL = 2048
NUM_EXPERTS = 16
TOP_K = 2

N_STREAMS = 4
BLOCK_T = 1024
HALF = BLOCK_T // N_STREAMS

_NT = (((1,), (1,)), ((), ()))


def _route_stream(xb, w_t, b_col, eps_blk):
    both_t = lax.dot_general(w_t, xb, _NT,
                             preferred_element_type=jnp.float32)
    both_t = both_t + b_col
    logits_t = both_t[:NUM_EXPERTS, :]
    nlogits_t = both_t[NUM_EXPERTS:, :]
    noisy = logits_t + eps_blk * jax.nn.softplus(nlogits_t)

    iota = lax.broadcasted_iota(jnp.int32, noisy.shape, 0)
    m1 = jnp.max(noisy, axis=0, keepdims=True)
    i1 = jnp.min(jnp.where(noisy == m1, iota, NUM_EXPERTS), axis=0,
                 keepdims=True)
    masked = jnp.where(iota == i1, -jnp.inf, noisy)
    m2 = jnp.max(masked, axis=0, keepdims=True)
    i2 = jnp.min(jnp.where(masked == m2, iota, NUM_EXPERTS), axis=0,
                 keepdims=True)
    keep = (iota == i1) | (iota == i2)
    z = jnp.where(keep, jnp.exp(noisy - m1), 0.0)
    out = z / jnp.sum(z, axis=0, keepdims=True)
    idx = jnp.concatenate([i1, i2], axis=0)
    return out, idx


def _router_body(*refs):
    x_refs = refs[:N_STREAMS]
    wr_ref, wn_ref, br_ref, bn_ref, eps_ref, out_ref, idx_ref = \
        refs[N_STREAMS:]
    w_t = jnp.concatenate([wr_ref[...], wn_ref[...]], axis=0)
    b_col = jnp.concatenate([br_ref[...], bn_ref[...]], axis=1).T
    for s in range(N_STREAMS):
        lo = s * HALF
        out_s, idx_s = _route_stream(x_refs[s][...], w_t, b_col,
                                     eps_ref[:, pl.ds(lo, HALF)])
        out_ref[:, pl.ds(lo, HALF)] = out_s
        idx_ref[:, pl.ds(lo, HALF)] = idx_s


def _x_spec(s):
    return pl.BlockSpec((HALF, D_MODEL), lambda i: (N_STREAMS * i + s, 0))


def kernel(x, W_route, b_route, W_noise, b_noise, eps):
    n_blocks = N_TOKENS // BLOCK_T
    wr_t = W_route.T
    wn_t = W_noise.T
    br = b_route.reshape(1, NUM_EXPERTS)
    bn = b_noise.reshape(1, NUM_EXPERTS)
    eps_t = eps.T
    out_shapes = (
        jax.ShapeDtypeStruct((NUM_EXPERTS, N_TOKENS), jnp.float32),
        jax.ShapeDtypeStruct((TOP_K, N_TOKENS), jnp.int32),
    )
    out_t, idx_t = pl.pallas_call(
        _router_body,
        grid=(n_blocks,),
        in_specs=[_x_spec(s) for s in range(N_STREAMS)] + [
            pl.BlockSpec((NUM_EXPERTS, D_MODEL), lambda i: (0, 0)),
            pl.BlockSpec((NUM_EXPERTS, D_MODEL), lambda i: (0, 0)),
            pl.BlockSpec((1, NUM_EXPERTS), lambda i: (0, 0)),
            pl.BlockSpec((1, NUM_EXPERTS), lambda i: (0, 0)),
            pl.BlockSpec((NUM_EXPERTS, BLOCK_T), lambda i: (0, i)),
        ],
        out_specs=(
            pl.BlockSpec((NUM_EXPERTS, BLOCK_T), lambda i: (0, i)),
            pl.BlockSpec((TOP_K, BLOCK_T), lambda i: (0, i)),
        ),
        out_shape=out_shapes,
    )(*([x] * N_STREAMS), wr_t, wn_t, br, bn, eps_t)
    return (out_t.T, idx_t.T)
